# baseline (speedup 1.0000x reference)
import functools
import jax
import jax.numpy as jnp
from jax import lax
from jax.experimental import pallas as pl
from jax.experimental.pallas import tpu as pltpu
from jax.experimental.pallas import tpu_sc as plsc

S, D = 8192, 1024
NC, NS = 2, 16
NW = NC * NS
ROWS_W = S // NW
CH = 32
NB = 3
NCHUNK = ROWS_W // CH


def _sc_body(w_hbm, o_hbm, buf, in_sems, out_sems):
    wid = lax.axis_index("s") * NC + lax.axis_index("c")
    base = wid * ROWS_W

    def in_copy(g, b):
        return pltpu.make_async_copy(
            w_hbm.at[pl.ds(base + g * CH, CH)], buf.at[b], in_sems.at[b])

    def out_copy(g, b):
        return pltpu.make_async_copy(
            buf.at[b], o_hbm.at[pl.ds(base + g * CH, CH)], out_sems.at[b])

    for b in range(NB):
        in_copy(b, b).start()
    for g in range(NCHUNK):
        b = g % NB
        in_copy(g, b).wait()
        out_copy(g, b).start()
        if g + NB < NCHUNK:
            out_copy(g, b).wait()
            in_copy(g + NB, b).start()
    for g in range(NCHUNK - NB, NCHUNK):
        if g >= 0:
            out_copy(g, g % NB).wait()


@jax.jit
def kernel(x, emb_weight):
    del x
    mesh = plsc.VectorSubcoreMesh(core_axis_name="c", subcore_axis_name="s")
    f = functools.partial(
        pl.kernel,
        out_type=jax.ShapeDtypeStruct((S, D), jnp.float32),
        mesh=mesh,
        scratch_types=[
            pltpu.VMEM((NB, CH, D), jnp.float32),
            pltpu.SemaphoreType.DMA((NB,)),
            pltpu.SemaphoreType.DMA((NB,)),
        ],
    )(_sc_body)
    return f(emb_weight)
